# Initial kernel scaffold; baseline (speedup 1.0000x reference)
import jax
import jax.numpy as jnp
from jax.experimental import pallas as pl


def kernel(feat, edge_index, W, b):
    raise NotImplementedError("write your pallas kernel here")


# speedup vs baseline: 21.6404x; 21.6404x over previous
import functools

import jax
import jax.numpy as jnp
from jax import lax
from jax.experimental import pallas as pl
from jax.experimental.pallas import tpu as pltpu
from jax.experimental.pallas import tpu_sc as plsc

_N = 10000
_D = 128
_C = 64
_K = 8
_ALPHA = 0.05

_NC = 2
_NS = 16
_NW = _NC * _NS
_NPAD = 10240
_RPT = _NPAD // _NS
_ZR = 64

_CB = 128
_NCH = 81
_EPAD = _NW * _NCH * _CB

_CK = [(1.0 - _ALPHA) * float(_K) ** (k + 1 - _K - 1) for k in range(_K)]
_CF = _ALPHA * sum(float(_K) ** (k + 1 - _K - 1) for k in range(_K))

_mesh = plsc.VectorSubcoreMesh(core_axis_name="c", subcore_axis_name="s")


def _deg_body(dst_hbm, out_hbm, degsh, dst_v, ones_v, zb):
    c = lax.axis_index("c")
    s = lax.axis_index("s")
    w = c * _NS + s
    ones16 = jnp.ones((16,), jnp.float32)
    zeros16 = jnp.zeros((16,), jnp.float32)
    for i in range(_CB // 16):
        ones_v[pl.ds(i * 16, 16)] = ones16
    for i in range(_RPT // 16):
        zb[pl.ds(i * 16, 16)] = zeros16
    pltpu.sync_copy(zb, degsh.at[pl.ds(s * _RPT, _RPT)])
    pltpu.sync_copy(dst_hbm.at[w], dst_v)
    plsc.subcore_barrier()
    for j in range(_NCH):
        pltpu.sync_copy(ones_v, degsh.at[dst_v.at[j]], add=True)
    plsc.subcore_barrier()
    pltpu.sync_copy(degsh.at[pl.ds(s * _RPT, _RPT)],
                    out_hbm.at[c, pl.ds(s * _RPT, _RPT)])


_deg_call = pl.kernel(
    _deg_body,
    out_type=jax.ShapeDtypeStruct((_NC, _NPAD), jnp.float32),
    mesh=_mesh,
    scratch_types=[
        pltpu.VMEM_SHARED((_NPAD,), jnp.float32),
        pltpu.VMEM((_NCH, _CB), jnp.int32),
        pltpu.VMEM((_CB,), jnp.float32),
        pltpu.VMEM((_RPT,), jnp.float32),
    ],
    compiler_params=pltpu.CompilerParams(use_tc_tiling_on_sc=False),
)


def _edge_body(z_hbm, src_hbm, dst_hbm, out_hbm,
               ssh, src_v, dst_v, g0, g1, zb, semg):
    c = lax.axis_index("c")
    s = lax.axis_index("s")
    w = c * _NS + s
    zeros16 = jnp.zeros((16,), jnp.float32)
    for r in range(_ZR):
        for q in range(_C // 16):
            zb[r, pl.ds(q * 16, 16)] = zeros16
    for blk in range(_RPT // _ZR):
        pltpu.sync_copy(zb, ssh.at[pl.ds(s * _RPT + blk * _ZR, _ZR)])
    pltpu.sync_copy(src_hbm.at[w], src_v)
    pltpu.sync_copy(dst_hbm.at[w], dst_v)
    plsc.subcore_barrier()
    gbufs = (g0, g1)
    pending = pltpu.async_copy(z_hbm.at[src_v.at[0]], g0, semg)
    for j in range(_NCH):
        pending.wait()
        cur = gbufs[j % 2]
        if j + 1 < _NCH:
            pending = pltpu.async_copy(
                z_hbm.at[src_v.at[j + 1]], gbufs[(j + 1) % 2], semg)
        pltpu.sync_copy(cur, ssh.at[dst_v.at[j]], add=True)
    plsc.subcore_barrier()
    pltpu.sync_copy(ssh.at[pl.ds(s * _RPT, _RPT)],
                    out_hbm.at[c, pl.ds(s * _RPT, _RPT)])


_edge_call = pl.kernel(
    _edge_body,
    out_type=jax.ShapeDtypeStruct((_NC, _NPAD, _C), jnp.float32),
    mesh=_mesh,
    scratch_types=[
        pltpu.VMEM_SHARED((_NPAD, _C), jnp.float32),
        pltpu.VMEM((_NCH, _CB), jnp.int32),
        pltpu.VMEM((_NCH, _CB), jnp.int32),
        pltpu.VMEM((_CB, _C), jnp.float32),
        pltpu.VMEM((_CB, _C), jnp.float32),
        pltpu.VMEM((_ZR, _C), jnp.float32),
        pltpu.SemaphoreType.DMA,
    ],
    compiler_params=pltpu.CompilerParams(use_tc_tiling_on_sc=False),
)


def _prep_body(feat_ref, wt_ref, degp_ref, y0_ref, z0_ref, dinv_ref, dinv2_ref):
    deg = jnp.maximum(degp_ref[0] + degp_ref[1], 1.0)
    dinv = lax.rsqrt(deg)
    y0 = jnp.dot(feat_ref[...], wt_ref[...], preferred_element_type=jnp.float32)
    y0_ref[...] = y0
    z0_ref[...] = y0 * dinv
    dinv_ref[...] = dinv
    dinv2_ref[...] = 1.0 / deg


def _prep_call(feat_pad, wt, degp3):
    return pl.pallas_call(
        _prep_body,
        out_shape=(
            jax.ShapeDtypeStruct((_NPAD, _C), jnp.float32),
            jax.ShapeDtypeStruct((_NPAD, _C), jnp.float32),
            jax.ShapeDtypeStruct((_NPAD, 1), jnp.float32),
            jax.ShapeDtypeStruct((_NPAD, 1), jnp.float32),
        ),
    )(feat_pad, wt, degp3)


def _scale_body(s_ref, dinv2_ref, z_ref):
    z_ref[...] = (s_ref[0] + s_ref[1]) * dinv2_ref[...]


def _scale_call(s, dinv2):
    return pl.pallas_call(
        _scale_body,
        out_shape=jax.ShapeDtypeStruct((_NPAD, _C), jnp.float32),
    )(s, dinv2)


def _final_body(*refs):
    s_refs = refs[:_K]
    y0_ref, dinv_ref, b_ref, out_ref = refs[_K:]
    t = _CK[0] * (s_refs[0][0] + s_refs[0][1])
    for k in range(1, _K):
        t = t + _CK[k] * (s_refs[k][0] + s_refs[k][1])
    out_ref[...] = t * dinv_ref[...] + _CF * y0_ref[...] + b_ref[...]


def _final_call(ss, y0, dinv, b2d):
    rb = _NPAD // 8
    s_spec = pl.BlockSpec((_NC, rb, _C), lambda i: (0, i, 0))
    return pl.pallas_call(
        _final_body,
        grid=(8,),
        in_specs=[s_spec] * _K + [
            pl.BlockSpec((rb, _C), lambda i: (i, 0)),
            pl.BlockSpec((rb, 1), lambda i: (i, 0)),
            pl.BlockSpec((1, _C), lambda i: (0, 0)),
        ],
        out_specs=pl.BlockSpec((rb, _C), lambda i: (i, 0)),
        out_shape=jax.ShapeDtypeStruct((_NPAD, _C), jnp.float32),
    )(*ss, y0, dinv, b2d)


@jax.jit
def kernel(feat, edge_index, W, b):
    feat_pad = jnp.zeros((_NPAD, _D), jnp.float32).at[:_N].set(feat)
    loop = jnp.arange(_N, dtype=jnp.int32)
    npadrows = _NPAD - _N
    e_in = edge_index.shape[1]
    padi = _N + (jnp.arange(_EPAD - _N - e_in, dtype=jnp.int32) % npadrows)
    src = jnp.concatenate([edge_index[0], loop, padi]).reshape(_NW, _NCH, _CB)
    dst = jnp.concatenate([edge_index[1], loop, padi]).reshape(_NW, _NCH, _CB)

    degp = _deg_call(dst)
    y0, z0, dinv, dinv2 = _prep_call(feat_pad, W.T, degp[:, :, None])

    ss = []
    z = z0
    for k in range(_K):
        s_k = _edge_call(z, src, dst)
        ss.append(s_k)
        if k < _K - 1:
            z = _scale_call(s_k, dinv2)

    out = _final_call(ss, y0, dinv, jnp.reshape(b, (1, _C)))
    return out[:_N]


# speedup vs baseline: 28.9455x; 1.3376x over previous
import functools

import jax
import jax.numpy as jnp
from jax import lax
from jax.experimental import pallas as pl
from jax.experimental.pallas import tpu as pltpu
from jax.experimental.pallas import tpu_sc as plsc

_N = 10000
_D = 128
_C = 64
_K = 8
_ALPHA = 0.05

_NC = 2
_NS = 16
_NW = _NC * _NS
_NPAD = 10240
_RPT = _NPAD // _NS
_ZR = 64

_CB = 128
_NCH = 81
_EPAD = _NW * _NCH * _CB

_CK = [(1.0 - _ALPHA) * float(_K) ** (k + 1 - _K - 1) for k in range(_K)]
_CF = _ALPHA * sum(float(_K) ** (k + 1 - _K - 1) for k in range(_K))

_mesh = plsc.VectorSubcoreMesh(core_axis_name="c", subcore_axis_name="s")


def _deg_body(dst_hbm, out_hbm, degsh, dst_v, ones_v, zb):
    c = lax.axis_index("c")
    s = lax.axis_index("s")
    w = c * _NS + s
    ones16 = jnp.ones((16,), jnp.float32)
    zeros16 = jnp.zeros((16,), jnp.float32)
    for i in range(_CB // 16):
        ones_v[pl.ds(i * 16, 16)] = ones16
    for i in range(_RPT // 16):
        zb[pl.ds(i * 16, 16)] = zeros16
    pltpu.sync_copy(zb, degsh.at[pl.ds(s * _RPT, _RPT)])
    pltpu.sync_copy(dst_hbm.at[w], dst_v)
    plsc.subcore_barrier()
    for j in range(_NCH):
        pltpu.sync_copy(ones_v, degsh.at[dst_v.at[j]], add=True)
    plsc.subcore_barrier()
    pltpu.sync_copy(degsh.at[pl.ds(s * _RPT, _RPT)],
                    out_hbm.at[c, pl.ds(s * _RPT, _RPT)])


_deg_call = pl.kernel(
    _deg_body,
    out_type=jax.ShapeDtypeStruct((_NC, _NPAD), jnp.float32),
    mesh=_mesh,
    scratch_types=[
        pltpu.VMEM_SHARED((_NPAD,), jnp.float32),
        pltpu.VMEM((_NCH, _CB), jnp.int32),
        pltpu.VMEM((_CB,), jnp.float32),
        pltpu.VMEM((_RPT,), jnp.float32),
    ],
    compiler_params=pltpu.CompilerParams(use_tc_tiling_on_sc=False),
)


def _edge_body(z_hbm, src_hbm, dst_hbm, out_hbm,
               ssh, src_v, dst_v, g0, g1, g2, g3, zb, semg, sems):
    c = lax.axis_index("c")
    s = lax.axis_index("s")
    w = c * _NS + s
    zeros16 = jnp.zeros((16,), jnp.float32)
    for r in range(_ZR):
        for q in range(_C // 16):
            zb[r, pl.ds(q * 16, 16)] = zeros16
    for blk in range(_RPT // _ZR):
        pltpu.sync_copy(zb, ssh.at[pl.ds(s * _RPT + blk * _ZR, _ZR)])
    pltpu.sync_copy(src_hbm.at[w], src_v)
    pltpu.sync_copy(dst_hbm.at[w], dst_v)
    plsc.subcore_barrier()
    gbufs = (g0, g1, g2, g3)

    def gfire(j):
        return pltpu.async_copy(z_hbm.at[src_v.at[j]], gbufs[j % 4], semg)

    def sfire(j):
        return pltpu.async_copy(gbufs[j % 4], ssh.at[dst_v.at[j]], sems,
                                add=True)

    gd = {j: gfire(j) for j in range(min(2, _NCH))}
    sd = {}
    for j in range(_NCH):
        gd.pop(j).wait()
        if j + 2 < _NCH:
            if j - 2 >= 0:
                sd.pop(j - 2).wait()
            gd[j + 2] = gfire(j + 2)
        sd[j] = sfire(j)
    for j in sorted(sd):
        sd.pop(j).wait()
    plsc.subcore_barrier()
    pltpu.sync_copy(ssh.at[pl.ds(s * _RPT, _RPT)],
                    out_hbm.at[c, pl.ds(s * _RPT, _RPT)])


_edge_call = pl.kernel(
    _edge_body,
    out_type=jax.ShapeDtypeStruct((_NC, _NPAD, _C), jnp.float32),
    mesh=_mesh,
    scratch_types=[
        pltpu.VMEM_SHARED((_NPAD, _C), jnp.float32),
        pltpu.VMEM((_NCH, _CB), jnp.int32),
        pltpu.VMEM((_NCH, _CB), jnp.int32),
        pltpu.VMEM((_CB, _C), jnp.float32),
        pltpu.VMEM((_CB, _C), jnp.float32),
        pltpu.VMEM((_CB, _C), jnp.float32),
        pltpu.VMEM((_CB, _C), jnp.float32),
        pltpu.VMEM((_ZR, _C), jnp.float32),
        pltpu.SemaphoreType.DMA,
        pltpu.SemaphoreType.DMA,
    ],
    compiler_params=pltpu.CompilerParams(use_tc_tiling_on_sc=False),
)


def _prep_body(feat_ref, wt_ref, degp_ref, y0_ref, z0_ref, dinv_ref, dinv2_ref):
    deg = jnp.maximum(degp_ref[0] + degp_ref[1], 1.0)
    dinv = lax.rsqrt(deg)
    y0 = jnp.dot(feat_ref[...], wt_ref[...], preferred_element_type=jnp.float32)
    y0_ref[...] = y0
    z0_ref[...] = y0 * dinv
    dinv_ref[...] = dinv
    dinv2_ref[...] = 1.0 / deg


def _prep_call(feat_pad, wt, degp3):
    return pl.pallas_call(
        _prep_body,
        out_shape=(
            jax.ShapeDtypeStruct((_NPAD, _C), jnp.float32),
            jax.ShapeDtypeStruct((_NPAD, _C), jnp.float32),
            jax.ShapeDtypeStruct((_NPAD, 1), jnp.float32),
            jax.ShapeDtypeStruct((_NPAD, 1), jnp.float32),
        ),
    )(feat_pad, wt, degp3)


def _scale_body(s_ref, dinv2_ref, z_ref):
    z_ref[...] = (s_ref[0] + s_ref[1]) * dinv2_ref[...]


def _scale_call(s, dinv2):
    return pl.pallas_call(
        _scale_body,
        out_shape=jax.ShapeDtypeStruct((_NPAD, _C), jnp.float32),
    )(s, dinv2)


def _final_body(*refs):
    s_refs = refs[:_K]
    y0_ref, dinv_ref, b_ref, out_ref = refs[_K:]
    t = _CK[0] * (s_refs[0][0] + s_refs[0][1])
    for k in range(1, _K):
        t = t + _CK[k] * (s_refs[k][0] + s_refs[k][1])
    out_ref[...] = t * dinv_ref[...] + _CF * y0_ref[...] + b_ref[...]


def _final_call(ss, y0, dinv, b2d):
    rb = _NPAD // 8
    s_spec = pl.BlockSpec((_NC, rb, _C), lambda i: (0, i, 0))
    return pl.pallas_call(
        _final_body,
        grid=(8,),
        in_specs=[s_spec] * _K + [
            pl.BlockSpec((rb, _C), lambda i: (i, 0)),
            pl.BlockSpec((rb, 1), lambda i: (i, 0)),
            pl.BlockSpec((1, _C), lambda i: (0, 0)),
        ],
        out_specs=pl.BlockSpec((rb, _C), lambda i: (i, 0)),
        out_shape=jax.ShapeDtypeStruct((_NPAD, _C), jnp.float32),
    )(*ss, y0, dinv, b2d)


@jax.jit
def kernel(feat, edge_index, W, b):
    feat_pad = jnp.zeros((_NPAD, _D), jnp.float32).at[:_N].set(feat)
    loop = jnp.arange(_N, dtype=jnp.int32)
    npadrows = _NPAD - _N
    e_in = edge_index.shape[1]
    padi = _N + (jnp.arange(_EPAD - _N - e_in, dtype=jnp.int32) % npadrows)
    src = jnp.concatenate([edge_index[0], loop, padi]).reshape(_NW, _NCH, _CB)
    dst = jnp.concatenate([edge_index[1], loop, padi]).reshape(_NW, _NCH, _CB)

    degp = _deg_call(dst)
    y0, z0, dinv, dinv2 = _prep_call(feat_pad, W.T, degp[:, :, None])

    ss = []
    z = z0
    for k in range(_K):
        s_k = _edge_call(z, src, dst)
        ss.append(s_k)
        if k < _K - 1:
            z = _scale_call(s_k, dinv2)

    out = _final_call(ss, y0, dinv, jnp.reshape(b, (1, _C)))
    return out[:_N]


# speedup vs baseline: 31.0122x; 1.0714x over previous
import functools

import jax
import jax.numpy as jnp
from jax import lax
from jax.experimental import pallas as pl
from jax.experimental.pallas import tpu as pltpu
from jax.experimental.pallas import tpu_sc as plsc

_N = 10000
_D = 128
_C = 64
_K = 8
_ALPHA = 0.05

_NC = 2
_NS = 16
_NW = _NC * _NS
_NPAD = 10240
_RPT = _NPAD // _NS
_ZR = 64

_CB = 128
_NCH = 81
_EPAD = _NW * _NCH * _CB

_CK = [(1.0 - _ALPHA) * float(_K) ** (k + 1 - _K - 1) for k in range(_K)]
_CF = _ALPHA * sum(float(_K) ** (k + 1 - _K - 1) for k in range(_K))

_mesh = plsc.VectorSubcoreMesh(core_axis_name="c", subcore_axis_name="s")


def _deg_body(dst_hbm, out_hbm, degsh, dst_v, ones_v, zb):
    c = lax.axis_index("c")
    s = lax.axis_index("s")
    w = c * _NS + s
    ones16 = jnp.ones((16,), jnp.float32)
    zeros16 = jnp.zeros((16,), jnp.float32)
    for i in range(_CB // 16):
        ones_v[pl.ds(i * 16, 16)] = ones16
    for i in range(_RPT // 16):
        zb[pl.ds(i * 16, 16)] = zeros16
    pltpu.sync_copy(zb, degsh.at[pl.ds(s * _RPT, _RPT)])
    pltpu.sync_copy(dst_hbm.at[w], dst_v)
    plsc.subcore_barrier()
    for j in range(_NCH):
        pltpu.sync_copy(ones_v, degsh.at[dst_v.at[j]], add=True)
    plsc.subcore_barrier()
    pltpu.sync_copy(degsh.at[pl.ds(s * _RPT, _RPT)],
                    out_hbm.at[c, pl.ds(s * _RPT, _RPT)])


_deg_call = pl.kernel(
    _deg_body,
    out_type=jax.ShapeDtypeStruct((_NC, _NPAD), jnp.float32),
    mesh=_mesh,
    scratch_types=[
        pltpu.VMEM_SHARED((_NPAD,), jnp.float32),
        pltpu.VMEM((_NCH, _CB), jnp.int32),
        pltpu.VMEM((_CB,), jnp.float32),
        pltpu.VMEM((_RPT,), jnp.float32),
    ],
    compiler_params=pltpu.CompilerParams(use_tc_tiling_on_sc=False),
)


def _edge_body(z_hbm, src_hbm, dst_hbm, out_hbm,
               ssh, src_v, dst_v, g0, g1, g2, g3, g4, g5, zb,
               semg, sems):
    c = lax.axis_index("c")
    s = lax.axis_index("s")
    w = c * _NS + s
    zeros16 = jnp.zeros((16,), jnp.float32)
    for r in range(_ZR):
        for q in range(_C // 16):
            zb[r, pl.ds(q * 16, 16)] = zeros16
    for blk in range(_RPT // _ZR):
        pltpu.sync_copy(zb, ssh.at[pl.ds(s * _RPT + blk * _ZR, _ZR)])
    pltpu.sync_copy(src_hbm.at[w], src_v)
    pltpu.sync_copy(dst_hbm.at[w], dst_v)
    plsc.subcore_barrier()
    gbufs = (g0, g1, g2, g3, g4, g5)
    nb = len(gbufs)
    gq = 3

    def gfire(j):
        return pltpu.async_copy(z_hbm.at[src_v.at[j]], gbufs[j % nb], semg)

    def sfire(j):
        return pltpu.async_copy(gbufs[j % nb], ssh.at[dst_v.at[j]], sems,
                                add=True)

    gd = {j: gfire(j) for j in range(min(gq, _NCH))}
    sd = {}
    for j in range(_NCH):
        gd.pop(j).wait()
        if j + gq < _NCH:
            if j + gq - nb >= 0:
                sd.pop(j + gq - nb).wait()
            gd[j + gq] = gfire(j + gq)
        sd[j] = sfire(j)
    for j in sorted(sd):
        sd.pop(j).wait()
    plsc.subcore_barrier()
    pltpu.sync_copy(ssh.at[pl.ds(s * _RPT, _RPT)],
                    out_hbm.at[c, pl.ds(s * _RPT, _RPT)])


_edge_call = pl.kernel(
    _edge_body,
    out_type=jax.ShapeDtypeStruct((_NC, _NPAD, _C), jnp.float32),
    mesh=_mesh,
    scratch_types=[
        pltpu.VMEM_SHARED((_NPAD, _C), jnp.float32),
        pltpu.VMEM((_NCH, _CB), jnp.int32),
        pltpu.VMEM((_NCH, _CB), jnp.int32),
        pltpu.VMEM((_CB, _C), jnp.float32),
        pltpu.VMEM((_CB, _C), jnp.float32),
        pltpu.VMEM((_CB, _C), jnp.float32),
        pltpu.VMEM((_CB, _C), jnp.float32),
        pltpu.VMEM((_CB, _C), jnp.float32),
        pltpu.VMEM((_CB, _C), jnp.float32),
        pltpu.VMEM((_ZR, _C), jnp.float32),
        pltpu.SemaphoreType.DMA,
        pltpu.SemaphoreType.DMA,
    ],
    compiler_params=pltpu.CompilerParams(use_tc_tiling_on_sc=False),
)


def _prep_body(feat_ref, wt_ref, degp_ref, y0_ref, z0_ref, dinv_ref, dinv2_ref):
    deg = jnp.maximum(degp_ref[0] + degp_ref[1], 1.0)
    dinv = lax.rsqrt(deg)
    y0 = jnp.dot(feat_ref[...], wt_ref[...], preferred_element_type=jnp.float32)
    y0_ref[...] = y0
    z0_ref[...] = y0 * dinv
    dinv_ref[...] = dinv
    dinv2_ref[...] = 1.0 / deg


def _prep_call(feat_pad, wt, degp3):
    return pl.pallas_call(
        _prep_body,
        out_shape=(
            jax.ShapeDtypeStruct((_NPAD, _C), jnp.float32),
            jax.ShapeDtypeStruct((_NPAD, _C), jnp.float32),
            jax.ShapeDtypeStruct((_NPAD, 1), jnp.float32),
            jax.ShapeDtypeStruct((_NPAD, 1), jnp.float32),
        ),
    )(feat_pad, wt, degp3)


def _scale_body(s_ref, dinv2_ref, z_ref):
    z_ref[...] = (s_ref[0] + s_ref[1]) * dinv2_ref[...]


def _scale_call(s, dinv2):
    return pl.pallas_call(
        _scale_body,
        out_shape=jax.ShapeDtypeStruct((_NPAD, _C), jnp.float32),
    )(s, dinv2)


def _final_body(*refs):
    s_refs = refs[:_K]
    y0_ref, dinv_ref, b_ref, out_ref = refs[_K:]
    t = _CK[0] * (s_refs[0][0] + s_refs[0][1])
    for k in range(1, _K):
        t = t + _CK[k] * (s_refs[k][0] + s_refs[k][1])
    out_ref[...] = t * dinv_ref[...] + _CF * y0_ref[...] + b_ref[...]


def _final_call(ss, y0, dinv, b2d):
    rb = _NPAD // 8
    s_spec = pl.BlockSpec((_NC, rb, _C), lambda i: (0, i, 0))
    return pl.pallas_call(
        _final_body,
        grid=(8,),
        in_specs=[s_spec] * _K + [
            pl.BlockSpec((rb, _C), lambda i: (i, 0)),
            pl.BlockSpec((rb, 1), lambda i: (i, 0)),
            pl.BlockSpec((1, _C), lambda i: (0, 0)),
        ],
        out_specs=pl.BlockSpec((rb, _C), lambda i: (i, 0)),
        out_shape=jax.ShapeDtypeStruct((_NPAD, _C), jnp.float32),
    )(*ss, y0, dinv, b2d)


@jax.jit
def kernel(feat, edge_index, W, b):
    feat_pad = jnp.zeros((_NPAD, _D), jnp.float32).at[:_N].set(feat)
    loop = jnp.arange(_N, dtype=jnp.int32)
    npadrows = _NPAD - _N
    e_in = edge_index.shape[1]
    padi = _N + (jnp.arange(_EPAD - _N - e_in, dtype=jnp.int32) % npadrows)
    src = jnp.concatenate([edge_index[0], loop, padi]).reshape(_NW, _NCH, _CB)
    dst = jnp.concatenate([edge_index[1], loop, padi]).reshape(_NW, _NCH, _CB)

    degp = _deg_call(dst)
    y0, z0, dinv, dinv2 = _prep_call(feat_pad, W.T, degp[:, :, None])

    ss = []
    z = z0
    for k in range(_K):
        s_k = _edge_call(z, src, dst)
        ss.append(s_k)
        if k < _K - 1:
            z = _scale_call(s_k, dinv2)

    out = _final_call(ss, y0, dinv, jnp.reshape(b, (1, _C)))
    return out[:_N]
